# Initial kernel scaffold; baseline (speedup 1.0000x reference)
import jax
import jax.numpy as jnp
from jax.experimental import pallas as pl


def kernel(pointcloud, params):
    raise NotImplementedError("write your pallas kernel here")


# speedup vs baseline: 1.0945x; 1.0945x over previous
import functools

import jax
import jax.numpy as jnp
from jax import lax
from jax.experimental import pallas as pl
from jax.experimental.pallas import tpu as pltpu
from jax.experimental.pallas import tpu_sc as plsc

_BN_EPS = 1e-5
_NPOINTS = [1024, 256, 64, 16]
_RADIUS = [[0.05, 0.1], [0.1, 0.2], [0.2, 0.4], [0.4, 0.8]]
_NSAMPLE = [[16, 32], [16, 32], [16, 32], [16, 32]]


@functools.lru_cache(maxsize=None)
def _fps_call(N, npoint):
    def body(x_ref, o_ref):
        x = x_ref[0]
        y = x_ref[1]
        z = x_ref[2]
        iota_n = lax.broadcasted_iota(jnp.int32, (8, N), 1)
        iota_p = lax.broadcasted_iota(jnp.int32, (8, npoint), 1)

        cx = x[:, 0:1]
        cy = y[:, 0:1]
        cz = z[:, 0:1]
        oh0 = (iota_p == 0).astype(jnp.float32)
        accx = cx * oh0
        accy = cy * oh0
        accz = cz * oh0
        dists0 = jnp.full((8, N), 1e10, jnp.float32)

        def step(j, st):
            dists, cx, cy, cz, accx, accy, accz = st
            d2 = (x - cx) ** 2 + (y - cy) ** 2 + (z - cz) ** 2
            dists = jnp.minimum(dists, d2)
            m = jnp.max(dists, axis=1, keepdims=True)
            nxt = jnp.min(jnp.where(dists == m, iota_n, N), axis=1,
                          keepdims=True)
            oh = (iota_n == nxt).astype(jnp.float32)
            ncx = jnp.sum(x * oh, axis=1, keepdims=True)
            ncy = jnp.sum(y * oh, axis=1, keepdims=True)
            ncz = jnp.sum(z * oh, axis=1, keepdims=True)
            ohj = (iota_p == j).astype(jnp.float32)
            return (dists, ncx, ncy, ncz,
                    accx + ncx * ohj, accy + ncy * ohj, accz + ncz * ohj)

        st = lax.fori_loop(1, npoint, step,
                           (dists0, cx, cy, cz, accx, accy, accz))
        o_ref[0] = st[4]
        o_ref[1] = st[5]
        o_ref[2] = st[6]

    return pl.pallas_call(
        body,
        out_shape=jax.ShapeDtypeStruct((3, 8, npoint), jnp.float32),
    )


@functools.lru_cache(maxsize=None)
def _sc_gather_call(V, D, Btot):
    NW = 32
    b_per_w = Btot // NW
    ch = b_per_w
    for c in (128, 96, 64, 48, 32, 16, 8):
        if b_per_w > 128 and b_per_w % c == 0:
            ch = c
            break
    nch = b_per_w // ch
    mesh = plsc.VectorSubcoreMesh(core_axis_name="c", subcore_axis_name="s")

    @functools.partial(
        pl.kernel,
        mesh=mesh,
        out_type=jax.ShapeDtypeStruct((Btot, D), jnp.float32),
        scratch_types=[
            pltpu.VMEM((ch,), jnp.int32),
            pltpu.VMEM((ch, D), jnp.float32),
            pltpu.SemaphoreType.DMA,
        ],
    )
    def k(table_hbm, idx_hbm, out_hbm, idx_v, rows_v, sem):
        wid = lax.axis_index("s") * 2 + lax.axis_index("c")
        base = wid * b_per_w

        def body(c, carry):
            off = base + c * ch
            pltpu.sync_copy(idx_hbm.at[pl.ds(off, ch)], idx_v)
            pltpu.async_copy(table_hbm.at[idx_v], rows_v, sem).wait()
            pltpu.sync_copy(rows_v, out_hbm.at[pl.ds(off, ch)])
            return carry

        lax.fori_loop(0, nch, body, 0)

    return k


def _gather_rows(table, idx_flat):
    V, D = table.shape
    return _sc_gather_call(V, D, idx_flat.shape[0])(table, idx_flat)


@functools.lru_cache(maxsize=None)
def _sa_mlp_call(B, K, Cin, S, C1, C2, C3):
    def body(x_ref, w1, b1, w2, b2, w3, b3, o_ref):
        x = x_ref[0, 0]
        h = jnp.maximum(
            jnp.dot(w1[...], x, preferred_element_type=jnp.float32) + b1[...],
            0.0)
        h = jnp.maximum(
            jnp.dot(w2[...], h, preferred_element_type=jnp.float32) + b2[...],
            0.0)
        y = jnp.maximum(
            jnp.dot(w3[...], h, preferred_element_type=jnp.float32) + b3[...],
            0.0)
        k = pl.program_id(1)

        @pl.when(k == 0)
        def _():
            o_ref[0] = y

        @pl.when(k > 0)
        def _():
            o_ref[0] = jnp.maximum(o_ref[0], y)

    def wspec(c_out, c_in):
        return pl.BlockSpec((c_out, c_in), lambda b, k: (0, 0))

    return pl.pallas_call(
        body,
        grid=(B, K),
        in_specs=[
            pl.BlockSpec((1, 1, Cin, S), lambda b, k: (b, k, 0, 0)),
            wspec(C1, Cin), wspec(C1, 1),
            wspec(C2, C1), wspec(C2, 1),
            wspec(C3, C2), wspec(C3, 1),
        ],
        out_specs=pl.BlockSpec((1, C3, S), lambda b, k: (b, 0, 0)),
        out_shape=jax.ShapeDtypeStruct((B, C3, S), jnp.float32),
    )


@functools.lru_cache(maxsize=None)
def _mlp2_call(B, Cin, M, C1, C2, relu1, relu2):
    def body(x_ref, w1, b1, w2, b2, o_ref):
        y = jnp.dot(w1[...], x_ref[0],
                    preferred_element_type=jnp.float32) + b1[...]
        if relu1:
            y = jnp.maximum(y, 0.0)
        y = jnp.dot(w2[...], y, preferred_element_type=jnp.float32) + b2[...]
        if relu2:
            y = jnp.maximum(y, 0.0)
        o_ref[0] = y

    def wspec(c_out, c_in):
        return pl.BlockSpec((c_out, c_in), lambda b: (0, 0))

    return pl.pallas_call(
        body,
        grid=(B,),
        in_specs=[
            pl.BlockSpec((1, Cin, M), lambda b: (b, 0, 0)),
            wspec(C1, Cin), wspec(C1, 1),
            wspec(C2, C1), wspec(C2, 1),
        ],
        out_specs=pl.BlockSpec((1, C2, M), lambda b: (b, 0, 0)),
        out_shape=jax.ShapeDtypeStruct((B, C2, M), jnp.float32),
    )


def _fold(layer):
    w, g, b = layer
    scale = g / jnp.sqrt(1.0 + _BN_EPS)
    return w * scale[:, None], b[:, None]


def _ball_query(xyz, new_xyz, radius, nsample):
    B, N, _ = xyz.shape
    S = new_xyz.shape[1]
    d2 = jnp.sum((new_xyz[:, :, None, :] - xyz[:, None, :, :]) ** 2, axis=-1)
    mask = d2 < radius * radius
    cum = jnp.cumsum(mask.astype(jnp.int32), axis=-1)
    slot = jnp.where(mask & (cum <= nsample), cum - 1, nsample)
    n_idx = jnp.broadcast_to(jnp.arange(N, dtype=jnp.int32), (B, S, N))
    buf = jnp.zeros((B, S, nsample + 1), jnp.int32)
    b_i = jnp.arange(B)[:, None, None]
    s_i = jnp.arange(S)[None, :, None]
    buf = buf.at[b_i, s_i, slot].set(n_idx)
    filled = jnp.minimum(cum[:, :, -1], nsample)
    first = buf[:, :, 0]
    k = jnp.arange(nsample, dtype=jnp.int32)
    return jnp.where(k[None, None, :] < jnp.maximum(filled, 1)[:, :, None],
                     buf[:, :, :nsample], first[:, :, None])


def _pad128(x):
    c = x.shape[-1]
    d = (-c) % 128
    if d:
        x = jnp.pad(x, [(0, 0)] * (x.ndim - 1) + [(0, d)])
    return x


def kernel(pointcloud, params):
    xyz = pointcloud[..., 0:3]
    B = xyz.shape[0]
    l_xyz = [xyz]
    l_feat = [None]

    for lvl in range(4):
        cur_xyz = l_xyz[lvl]
        feats = l_feat[lvl]
        N = cur_xyz.shape[1]
        npoint = _NPOINTS[lvl]

        fps_out = _fps_call(N, npoint)(jnp.transpose(cur_xyz, (2, 0, 1)))
        new_xyz = jnp.transpose(fps_out, (1, 2, 0))

        if feats is None:
            tab = cur_xyz
            C = 3
        else:
            tab = jnp.concatenate(
                [cur_xyz, jnp.transpose(feats, (0, 2, 1))], axis=-1)
            C = tab.shape[-1]
        tab = _pad128(tab)
        D = tab.shape[-1]
        tabf = tab.reshape(B * N, D)
        boff = (jnp.arange(B, dtype=jnp.int32) * N)[:, None, None]

        outs = []
        for s in range(2):
            K = _NSAMPLE[lvl][s]
            idx = _ball_query(cur_xyz, new_xyz, _RADIUS[lvl][s], K)
            gidx = (idx + boff).reshape(-1)
            rows = _gather_rows(tabf, gidx).reshape(B, npoint, K, D)
            gx = rows[..., :3] - new_xyz[:, :, None, :]
            if feats is None:
                xin = gx
            else:
                xin = jnp.concatenate([gx, rows[..., 3:C]], axis=-1)
            cin = xin.shape[-1]
            xk = jnp.transpose(xin, (0, 2, 3, 1))
            lw = [_fold(l) for l in params['sa'][lvl][s]]
            c1, c2, c3 = (lw[0][0].shape[0], lw[1][0].shape[0],
                          lw[2][0].shape[0])
            y = _sa_mlp_call(B, K, cin, npoint, c1, c2, c3)(
                xk, lw[0][0], lw[0][1], lw[1][0], lw[1][1],
                lw[2][0], lw[2][1])
            outs.append(y)
        l_xyz.append(new_xyz)
        l_feat.append(jnp.concatenate(outs, axis=1))

    for i in range(-1, -5, -1):
        unknown = l_xyz[i - 1]
        known = l_xyz[i]
        unk_f = l_feat[i - 1]
        kn_f = l_feat[i]
        n = unknown.shape[1]
        m = known.shape[1]
        d2 = jnp.sum((unknown[:, :, None, :] - known[:, None, :, :]) ** 2,
                     axis=-1)
        neg, idx = lax.top_k(-d2, 3)
        dist = -neg
        dist_recip = 1.0 / (dist + 1e-8)
        norm = jnp.sum(dist_recip, axis=2, keepdims=True)
        weight = dist_recip / norm

        C = kn_f.shape[1]
        tabf = jnp.transpose(kn_f, (0, 2, 1)).reshape(B * m, C)
        boff = (jnp.arange(B, dtype=jnp.int32) * m)[:, None, None]
        gidx = (idx.astype(jnp.int32) + boff).reshape(-1)
        rows = _gather_rows(tabf, gidx).reshape(B, n, 3, C)
        interp = jnp.sum(rows * weight[..., None], axis=2)
        interp = jnp.transpose(interp, (0, 2, 1))
        nf = interp if unk_f is None else jnp.concatenate([interp, unk_f],
                                                          axis=1)
        lw = [_fold(l) for l in params['fp'][i]]
        c1, c2 = lw[0][0].shape[0], lw[1][0].shape[0]
        l_feat[i - 1] = _mlp2_call(B, nf.shape[1], n, c1, c2, True, True)(
            nf, lw[0][0], lw[0][1], lw[1][0], lw[1][1])

    x = l_feat[0]
    lw = [_fold(l) for l in params['cls']]
    c1, c2 = lw[0][0].shape[0], lw[1][0].shape[0]
    x = _mlp2_call(B, x.shape[1], x.shape[2], c1, c2, True, False)(
        x, lw[0][0], lw[0][1], lw[1][0], lw[1][1])
    return jnp.transpose(x, (0, 2, 1))


# speedup vs baseline: 28.1147x; 25.6864x over previous
import functools

import jax
import jax.numpy as jnp
from jax import lax
from jax.experimental import pallas as pl
from jax.experimental.pallas import tpu as pltpu
from jax.experimental.pallas import tpu_sc as plsc

_BN_EPS = 1e-5
_NPOINTS = [1024, 256, 64, 16]
_RADIUS = [[0.05, 0.1], [0.1, 0.2], [0.2, 0.4], [0.4, 0.8]]
_NSAMPLE = [[16, 32], [16, 32], [16, 32], [16, 32]]


@functools.lru_cache(maxsize=None)
def _fps_call(N, npoint):
    def body(x_ref, o_ref):
        x = x_ref[0]
        y = x_ref[1]
        z = x_ref[2]
        iota_n = lax.broadcasted_iota(jnp.int32, (8, N), 1)
        iota_p = lax.broadcasted_iota(jnp.int32, (8, npoint), 1)

        cx = x[:, 0:1]
        cy = y[:, 0:1]
        cz = z[:, 0:1]
        oh0 = (iota_p == 0).astype(jnp.float32)
        accx = cx * oh0
        accy = cy * oh0
        accz = cz * oh0
        dists0 = jnp.full((8, N), 1e10, jnp.float32)

        def step(j, st):
            dists, cx, cy, cz, accx, accy, accz = st
            d2 = (x - cx) ** 2 + (y - cy) ** 2 + (z - cz) ** 2
            dists = jnp.minimum(dists, d2)
            m = jnp.max(dists, axis=1, keepdims=True)
            nxt = jnp.min(jnp.where(dists == m, iota_n, N), axis=1,
                          keepdims=True)
            oh = (iota_n == nxt).astype(jnp.float32)
            ncx = jnp.sum(x * oh, axis=1, keepdims=True)
            ncy = jnp.sum(y * oh, axis=1, keepdims=True)
            ncz = jnp.sum(z * oh, axis=1, keepdims=True)
            ohj = (iota_p == j).astype(jnp.float32)
            return (dists, ncx, ncy, ncz,
                    accx + ncx * ohj, accy + ncy * ohj, accz + ncz * ohj)

        st = lax.fori_loop(1, npoint, step,
                           (dists0, cx, cy, cz, accx, accy, accz))
        o_ref[0] = st[4]
        o_ref[1] = st[5]
        o_ref[2] = st[6]

    return pl.pallas_call(
        body,
        out_shape=jax.ShapeDtypeStruct((3, 8, npoint), jnp.float32),
    )


@functools.lru_cache(maxsize=None)
def _sc_gather_call(V, D, Btot):
    NW = 32
    b_per_w = Btot // NW
    ch = b_per_w
    for c in (128, 96, 64, 48, 32, 16, 8):
        if b_per_w > 128 and b_per_w % c == 0:
            ch = c
            break
    nch = b_per_w // ch
    mesh = plsc.VectorSubcoreMesh(core_axis_name="c", subcore_axis_name="s")

    @functools.partial(
        pl.kernel,
        mesh=mesh,
        out_type=jax.ShapeDtypeStruct((Btot, D), jnp.float32),
        scratch_types=[
            pltpu.VMEM((ch,), jnp.int32),
            pltpu.VMEM((ch, D), jnp.float32),
            pltpu.SemaphoreType.DMA,
        ],
    )
    def k(table_hbm, idx_hbm, out_hbm, idx_v, rows_v, sem):
        wid = lax.axis_index("s") * 2 + lax.axis_index("c")
        base = wid * b_per_w

        def body(c, carry):
            off = base + c * ch
            pltpu.sync_copy(idx_hbm.at[pl.ds(off, ch)], idx_v)
            pltpu.async_copy(table_hbm.at[idx_v], rows_v, sem).wait()
            pltpu.sync_copy(rows_v, out_hbm.at[pl.ds(off, ch)])
            return carry

        lax.fori_loop(0, nch, body, 0)

    return k


def _gather_rows(table, idx_flat):
    V, D = table.shape
    return _sc_gather_call(V, D, idx_flat.shape[0])(table, idx_flat)


@functools.lru_cache(maxsize=None)
def _sa_mlp_call(B, K, Cin, S, C1, C2, C3):
    def body(x_ref, w1, b1, w2, b2, w3, b3, o_ref):
        x = x_ref[0, 0]
        h = jnp.maximum(
            jnp.dot(w1[...], x, preferred_element_type=jnp.float32) + b1[...],
            0.0)
        h = jnp.maximum(
            jnp.dot(w2[...], h, preferred_element_type=jnp.float32) + b2[...],
            0.0)
        y = jnp.maximum(
            jnp.dot(w3[...], h, preferred_element_type=jnp.float32) + b3[...],
            0.0)
        k = pl.program_id(1)

        @pl.when(k == 0)
        def _():
            o_ref[0] = y

        @pl.when(k > 0)
        def _():
            o_ref[0] = jnp.maximum(o_ref[0], y)

    def wspec(c_out, c_in):
        return pl.BlockSpec((c_out, c_in), lambda b, k: (0, 0))

    return pl.pallas_call(
        body,
        grid=(B, K),
        in_specs=[
            pl.BlockSpec((1, 1, Cin, S), lambda b, k: (b, k, 0, 0)),
            wspec(C1, Cin), wspec(C1, 1),
            wspec(C2, C1), wspec(C2, 1),
            wspec(C3, C2), wspec(C3, 1),
        ],
        out_specs=pl.BlockSpec((1, C3, S), lambda b, k: (b, 0, 0)),
        out_shape=jax.ShapeDtypeStruct((B, C3, S), jnp.float32),
    )


@functools.lru_cache(maxsize=None)
def _mlp2_call(B, Cin, M, C1, C2, relu1, relu2):
    def body(x_ref, w1, b1, w2, b2, o_ref):
        y = jnp.dot(w1[...], x_ref[0],
                    preferred_element_type=jnp.float32) + b1[...]
        if relu1:
            y = jnp.maximum(y, 0.0)
        y = jnp.dot(w2[...], y, preferred_element_type=jnp.float32) + b2[...]
        if relu2:
            y = jnp.maximum(y, 0.0)
        o_ref[0] = y

    def wspec(c_out, c_in):
        return pl.BlockSpec((c_out, c_in), lambda b: (0, 0))

    return pl.pallas_call(
        body,
        grid=(B,),
        in_specs=[
            pl.BlockSpec((1, Cin, M), lambda b: (b, 0, 0)),
            wspec(C1, Cin), wspec(C1, 1),
            wspec(C2, C1), wspec(C2, 1),
        ],
        out_specs=pl.BlockSpec((1, C2, M), lambda b: (b, 0, 0)),
        out_shape=jax.ShapeDtypeStruct((B, C2, M), jnp.float32),
    )


def _fold(layer):
    w, g, b = layer
    scale = g / jnp.sqrt(1.0 + _BN_EPS)
    return w * scale[:, None], b[:, None]


@functools.lru_cache(maxsize=None)
def _ballq_call(B, N, S, TS, r1, K1, r2, K2):
    def body(x_ref, q_ref, o1_ref, o2_ref):
        xt = x_ref[0]
        q = q_ref[0]
        iota = lax.broadcasted_iota(jnp.int32, (TS, N), 1)
        dx = q[:, 0:1] - xt[0:1, :]
        dy = q[:, 1:2] - xt[1:2, :]
        dz = q[:, 2:3] - xt[2:3, :]
        d2 = dx * dx + dy * dy + dz * dz

        for r, K, oref in ((r1, K1, o1_ref), (r2, K2, o2_ref)):
            avail = d2 < r * r
            cols = []
            first = None
            for k in range(K):
                nxt = jnp.min(jnp.where(avail, iota, N), axis=1,
                              keepdims=True)
                if k == 0:
                    first = nxt
                    cols.append(nxt)
                else:
                    cols.append(jnp.where(nxt == N, first, nxt))
                avail = avail & (iota != nxt)
            oref[0] = jnp.concatenate(cols, axis=1)

    return pl.pallas_call(
        body,
        grid=(B, S // TS),
        in_specs=[
            pl.BlockSpec((1, 3, N), lambda b, s: (b, 0, 0)),
            pl.BlockSpec((1, TS, 3), lambda b, s: (b, s, 0)),
        ],
        out_specs=[
            pl.BlockSpec((1, TS, K1), lambda b, s: (b, s, 0)),
            pl.BlockSpec((1, TS, K2), lambda b, s: (b, s, 0)),
        ],
        out_shape=[
            jax.ShapeDtypeStruct((B, S, K1), jnp.int32),
            jax.ShapeDtypeStruct((B, S, K2), jnp.int32),
        ],
    )


def _pad128(x):
    c = x.shape[-1]
    d = (-c) % 128
    if d:
        x = jnp.pad(x, [(0, 0)] * (x.ndim - 1) + [(0, d)])
    return x


def kernel(pointcloud, params):
    xyz = pointcloud[..., 0:3]
    B = xyz.shape[0]
    l_xyz = [xyz]
    l_feat = [None]

    for lvl in range(4):
        cur_xyz = l_xyz[lvl]
        feats = l_feat[lvl]
        N = cur_xyz.shape[1]
        npoint = _NPOINTS[lvl]

        xyz_sb = jnp.transpose(cur_xyz, (2, 0, 1))
        fps_out = _fps_call(N, npoint)(xyz_sb)
        new_xyz = jnp.transpose(fps_out, (1, 2, 0))

        ts = min(npoint, 256)
        idx_pair = _ballq_call(
            B, N, npoint, ts, _RADIUS[lvl][0], _NSAMPLE[lvl][0],
            _RADIUS[lvl][1], _NSAMPLE[lvl][1])(
                jnp.transpose(xyz_sb, (1, 0, 2)), new_xyz)

        if feats is None:
            tab = cur_xyz
            C = 3
        else:
            tab = jnp.concatenate(
                [cur_xyz, jnp.transpose(feats, (0, 2, 1))], axis=-1)
            C = tab.shape[-1]
        tab = _pad128(tab)
        D = tab.shape[-1]
        tabf = tab.reshape(B * N, D)
        boff = (jnp.arange(B, dtype=jnp.int32) * N)[:, None, None]

        outs = []
        for s in range(2):
            K = _NSAMPLE[lvl][s]
            idx = idx_pair[s]
            gidx = (idx + boff).reshape(-1)
            rows = _gather_rows(tabf, gidx).reshape(B, npoint, K, D)
            gx = rows[..., :3] - new_xyz[:, :, None, :]
            if feats is None:
                xin = gx
            else:
                xin = jnp.concatenate([gx, rows[..., 3:C]], axis=-1)
            cin = xin.shape[-1]
            xk = jnp.transpose(xin, (0, 2, 3, 1))
            lw = [_fold(l) for l in params['sa'][lvl][s]]
            c1, c2, c3 = (lw[0][0].shape[0], lw[1][0].shape[0],
                          lw[2][0].shape[0])
            y = _sa_mlp_call(B, K, cin, npoint, c1, c2, c3)(
                xk, lw[0][0], lw[0][1], lw[1][0], lw[1][1],
                lw[2][0], lw[2][1])
            outs.append(y)
        l_xyz.append(new_xyz)
        l_feat.append(jnp.concatenate(outs, axis=1))

    for i in range(-1, -5, -1):
        unknown = l_xyz[i - 1]
        known = l_xyz[i]
        unk_f = l_feat[i - 1]
        kn_f = l_feat[i]
        n = unknown.shape[1]
        m = known.shape[1]
        d2 = jnp.sum((unknown[:, :, None, :] - known[:, None, :, :]) ** 2,
                     axis=-1)
        neg, idx = lax.top_k(-d2, 3)
        dist = -neg
        dist_recip = 1.0 / (dist + 1e-8)
        norm = jnp.sum(dist_recip, axis=2, keepdims=True)
        weight = dist_recip / norm

        C = kn_f.shape[1]
        tabf = jnp.transpose(kn_f, (0, 2, 1)).reshape(B * m, C)
        boff = (jnp.arange(B, dtype=jnp.int32) * m)[:, None, None]
        gidx = (idx.astype(jnp.int32) + boff).reshape(-1)
        rows = _gather_rows(tabf, gidx).reshape(B, n, 3, C)
        interp = jnp.sum(rows * weight[..., None], axis=2)
        interp = jnp.transpose(interp, (0, 2, 1))
        nf = interp if unk_f is None else jnp.concatenate([interp, unk_f],
                                                          axis=1)
        lw = [_fold(l) for l in params['fp'][i]]
        c1, c2 = lw[0][0].shape[0], lw[1][0].shape[0]
        l_feat[i - 1] = _mlp2_call(B, nf.shape[1], n, c1, c2, True, True)(
            nf, lw[0][0], lw[0][1], lw[1][0], lw[1][1])

    x = l_feat[0]
    lw = [_fold(l) for l in params['cls']]
    c1, c2 = lw[0][0].shape[0], lw[1][0].shape[0]
    x = _mlp2_call(B, x.shape[1], x.shape[2], c1, c2, True, False)(
        x, lw[0][0], lw[0][1], lw[1][0], lw[1][1])
    return jnp.transpose(x, (0, 2, 1))
